# baseline (device time: 46624 ns/iter reference)
import jax
import jax.numpy as jnp
from jax import lax
from jax.experimental import pallas as pl
from jax.experimental.pallas import tpu as pltpu

N_DEV = 4
WIN = 128
QB = 128
KB = 3 * WIN


def kernel(x, Wq, K_ext, V_ext, Wo):
    B, Sq, HD = x.shape
    _, Skv, Hq, Dh = K_ext.shape
    Dm = Wq.shape[1]
    Sh = Skv + 2 * WIN
    NQB = Sq // QB

    def body(x_ref, wq_ref, k_ref, v_ref, wo_ref, out_ref,
             kbuf, vbuf, send_sems, recv_sems):
        my = lax.axis_index("i")
        left = lax.rem(my + N_DEV - 1, N_DEV)
        right = lax.rem(my + 1, N_DEV)

        barrier_sem = pltpu.get_barrier_semaphore()
        for nbr in (left, right):
            pl.semaphore_signal(
                barrier_sem, inc=1,
                device_id=(nbr,), device_id_type=pl.DeviceIdType.MESH,
            )
        pl.semaphore_wait(barrier_sem, 2)

        to_left, to_right = [], []
        for idx, (src, dbuf) in enumerate([(k_ref, kbuf), (v_ref, vbuf)]):
            r = pltpu.make_async_remote_copy(
                src_ref=src.at[:, pl.ds(0, WIN)],
                dst_ref=dbuf.at[:, pl.ds(WIN + Skv, WIN)],
                send_sem=send_sems.at[idx],
                recv_sem=recv_sems.at[idx],
                device_id=(left,), device_id_type=pl.DeviceIdType.MESH,
            )
            r.start()
            to_left.append(r)
        for idx, (src, dbuf) in enumerate([(k_ref, kbuf), (v_ref, vbuf)], 2):
            r = pltpu.make_async_remote_copy(
                src_ref=src.at[:, pl.ds(Skv - WIN, WIN)],
                dst_ref=dbuf.at[:, pl.ds(0, WIN)],
                send_sem=send_sems.at[idx],
                recv_sem=recv_sems.at[idx],
                device_id=(right,), device_id_type=pl.DeviceIdType.MESH,
            )
            r.start()
            to_right.append(r)

        kbuf[:, WIN:WIN + Skv] = k_ref[...]
        vbuf[:, WIN:WIN + Skv] = v_ref[...]

        xr = x_ref[...].reshape(B * Sq, HD)
        Q = lax.dot(xr, wq_ref[...], preferred_element_type=jnp.float32)

        qi = lax.broadcasted_iota(jnp.int32, (Sq, Sh), 0)
        ki = lax.broadcasted_iota(jnp.int32, (Sq, Sh), 1)
        ki_g = my * Skv - WIN + ki
        valid = (jnp.abs(qi - ki + WIN) <= WIN) & (ki_g >= 0) & (ki_g < N_DEV * Skv)
        neg = jnp.float32(-1e9)

        def attn_block(b, h, qb, kb_val, vb_val):
            q = Q[b * Sq + qb * QB:b * Sq + (qb + 1) * QB,
                  h * Dh:(h + 1) * Dh]
            k = kb_val[b, qb * QB:qb * QB + KB, h, :]
            s = lax.dot_general(
                q, k, (((1,), (1,)), ((), ())),
                preferred_element_type=jnp.float32,
            ) * 0.125
            vmask = valid[qb * QB:(qb + 1) * QB, qb * QB:qb * QB + KB]
            p = jnp.exp(jnp.where(vmask, s, neg))
            denom = jnp.sum(p, axis=-1, keepdims=True)
            v = vb_val[b, qb * QB:qb * QB + KB, h, :]
            return lax.dot(p, v, preferred_element_type=jnp.float32) / denom

        ctx = {}

        kb_val = kbuf[...]
        vb_val = vbuf[...]
        for b in range(B):
            for h in range(Hq):
                for qb in (1, 2):
                    ctx[b, h, qb] = attn_block(b, h, qb, kb_val, vb_val)

        for r in to_right:
            r.wait_recv()
        kb_val = kbuf[...]
        vb_val = vbuf[...]
        for b in range(B):
            for h in range(Hq):
                ctx[b, h, 0] = attn_block(b, h, 0, kb_val, vb_val)

        for r in to_left:
            r.wait_recv()
        kb_val = kbuf[...]
        vb_val = vbuf[...]
        for b in range(B):
            for h in range(Hq):
                ctx[b, h, NQB - 1] = attn_block(b, h, NQB - 1, kb_val, vb_val)

        for r in to_left + to_right:
            r.wait_send()

        for b in range(B):
            cb = jnp.concatenate(
                [jnp.concatenate([ctx[b, h, qb] for qb in range(NQB)], axis=0)
                 for h in range(Hq)], axis=1)
            out_ref[b] = lax.dot(cb, wo_ref[...],
                                 preferred_element_type=jnp.float32)

    return pl.pallas_call(
        body,
        out_shape=jax.ShapeDtypeStruct((B, Sq, HD), jnp.float32),
        in_specs=[pl.BlockSpec(memory_space=pltpu.VMEM)] * 5,
        out_specs=pl.BlockSpec(memory_space=pltpu.VMEM),
        scratch_shapes=[
            pltpu.VMEM((B, Sh, Hq, Dh), jnp.float32),
            pltpu.VMEM((B, Sh, Hq, Dh), jnp.float32),
            pltpu.SemaphoreType.DMA((4,)),
            pltpu.SemaphoreType.DMA((4,)),
        ],
        compiler_params=pltpu.CompilerParams(collective_id=0),
    )(x, Wq, K_ext, V_ext, Wo)
